# baseline (speedup 1.0000x reference)
import functools
import math

import jax
import jax.numpy as jnp
from jax import lax
from jax.experimental import pallas as pl
from jax.experimental.pallas import tpu as pltpu
from jax.experimental.pallas import tpu_sc as plsc

_WN = 1000
_TN = 5000
_L = 4
_A = 128
_K = _TN * _L
_LANES = 16
_KCH = _K // _LANES
_NTILES = 32
_RPT = 32
_LN3 = math.log(3.0)


def _sc_body(wf_hbm, par_hbm, tau_hbm, out_hbm,
             wf_v, par_v, tau_v, row00, row01, row10, row11, ac_v,
             sem_in, sem0, sem1):
    cid = lax.axis_index("c")
    sid = lax.axis_index("s")
    wid = sid * 2 + cid
    base = wid * _RPT

    pltpu.sync_copy(par_hbm, par_v)
    pltpu.sync_copy(tau_hbm, tau_v)
    pltpu.sync_copy(wf_hbm.at[pl.ds(wid * _A * _RPT, _A * _RPT)], wf_v)

    bvec = par_v[pl.ds(_A, _LANES)]

    for h in range(_RPT // _LANES):
        zvec = bvec
        for ch in range(_A // _LANES):
            pv = par_v[pl.ds(ch * _LANES, _LANES)]
            for j in range(_LANES):
                f = ch * _LANES + j
                zvec = zvec + wf_v[pl.ds(f * _RPT + h * _LANES, _LANES)] * pv[j]
        ac_v[pl.ds(h * _LANES, _LANES)] = zvec + _LN3
        ac_v[pl.ds(_RPT + h * _LANES, _LANES)] = 1.0 / (3.0 * (1.0 + jnp.exp(zvec)))

    bufs = ((row00, row01), (row10, row11))
    sems = (sem0, sem1)
    handles = [None, None]
    for p in range(_RPT // 2):
        slot = p % 2
        bufa, bufb = bufs[slot]
        if handles[slot] is not None:
            for hd in handles[slot]:
                hd.wait()
        sc = []
        for r in range(2):
            w = 2 * p + r
            h, ln = w // _LANES, w % _LANES
            av = ac_v[pl.ds(h * _LANES, _LANES)][ln]
            cv = ac_v[pl.ds(_RPT + h * _LANES, _LANES)][ln]
            sc.append((av, cv))
        (a0, c0), (a1, c1) = sc

        @plsc.parallel_loop(0, _K, _LANES, unroll=16)
        def _row(o, bufa=bufa, bufb=bufb, a0=a0, c0=c0, a1=a1, c1=c1):
            t = tau_v[pl.ds(o, _LANES)]
            bufa[pl.ds(o, _LANES)] = c0 * jnp.exp(t * a0)
            bufb[pl.ds(o, _LANES)] = c1 * jnp.exp(t * a1)

        i0 = jnp.minimum(base + 2 * p, _WN - 1)
        i1 = jnp.minimum(base + 2 * p + 1, _WN - 1)
        handles[slot] = (
            pltpu.async_copy(bufa, out_hbm.at[pl.ds(i0 * _K, _K)], sems[slot]),
            pltpu.async_copy(bufb, out_hbm.at[pl.ds(i1 * _K, _K)], sems[slot]),
        )
    for hds in handles:
        for hd in hds:
            hd.wait()


@jax.jit
def _run(wf, par, tau):
    mesh = plsc.VectorSubcoreMesh(core_axis_name="c", subcore_axis_name="s")
    f = functools.partial(
        pl.kernel,
        mesh=mesh,
        out_type=jax.ShapeDtypeStruct((_WN * _K,), jnp.float32),
        scratch_types=[
            pltpu.VMEM((_A * _RPT,), jnp.float32),
            pltpu.VMEM((_A + _LANES,), jnp.float32),
            pltpu.VMEM((_K,), jnp.float32),
            pltpu.VMEM((_K,), jnp.float32),
            pltpu.VMEM((_K,), jnp.float32),
            pltpu.VMEM((_K,), jnp.float32),
            pltpu.VMEM((_K,), jnp.float32),
            pltpu.VMEM((2 * _RPT,), jnp.float32),
            pltpu.SemaphoreType.DMA,
            pltpu.SemaphoreType.DMA,
            pltpu.SemaphoreType.DMA,
        ],
    )(_sc_body)
    return f(wf, par, tau)


def kernel(inputs, W, b):
    wf = inputs[:_WN, :_A]
    pad = jnp.broadcast_to(wf[_WN - 1], (_NTILES * _RPT - _WN, _A))
    wf = jnp.concatenate([wf, pad])
    wf = wf.reshape(_NTILES, _RPT, _A).transpose(0, 2, 1).reshape(_NTILES * _A * _RPT)
    tau = inputs[_WN:, :_L].reshape(_K)
    par = jnp.concatenate([W[:, 0], jnp.broadcast_to(b, (_LANES,))])
    out = _run(wf, par, tau)
    return out.reshape(_WN, _TN, _L)


# speedup vs baseline: 7.0643x; 7.0643x over previous
import functools
import math

import jax
import jax.numpy as jnp
from jax import lax
from jax.experimental import pallas as pl
from jax.experimental.pallas import tpu as pltpu
from jax.experimental.pallas import tpu_sc as plsc

_WN = 1000
_TN = 5000
_L = 4
_A = 128
_K = _TN * _L
_LANES = 16
_KCH = _K // _LANES
_NTILES = 32
_RPT = 32
_LN3 = math.log(3.0)


def _sc_body(wf_hbm, par_hbm, tau_hbm, out_hbm,
             wf_v, par_v, tau_v, row00, row01, row10, row11, ac_v,
             sem_in, sem0, sem1):
    cid = lax.axis_index("c")
    sid = lax.axis_index("s")
    wid = sid * 2 + cid
    base = wid * _RPT

    pltpu.sync_copy(par_hbm, par_v)
    pltpu.sync_copy(tau_hbm, tau_v)
    pltpu.sync_copy(wf_hbm.at[wid], wf_v)

    bvec = par_v[pl.ds(_A, _LANES)]

    for h in range(_RPT // _LANES):
        zvec = bvec
        for ch in range(_A // _LANES):
            pv = par_v[pl.ds(ch * _LANES, _LANES)]
            for j in range(_LANES):
                f = ch * _LANES + j
                zvec = zvec + wf_v[pl.ds(f * _RPT + h * _LANES, _LANES)] * pv[j]
        ac_v[pl.ds(h * _LANES, _LANES)] = zvec + _LN3
        ac_v[pl.ds(_RPT + h * _LANES, _LANES)] = 1.0 / (3.0 * (1.0 + jnp.exp(zvec)))

    bufs = ((row00, row01), (row10, row11))
    sems = (sem0, sem1)
    handles = [None, None]
    for p in range(_RPT // 2):
        slot = p % 2
        bufa, bufb = bufs[slot]
        if handles[slot] is not None:
            for hd in handles[slot]:
                hd.wait()
        sc = []
        for r in range(2):
            w = 2 * p + r
            h, ln = w // _LANES, w % _LANES
            av = ac_v[pl.ds(h * _LANES, _LANES)][ln]
            cv = ac_v[pl.ds(_RPT + h * _LANES, _LANES)][ln]
            sc.append((av, cv))
        (a0, c0), (a1, c1) = sc

        @plsc.parallel_loop(0, _K, _LANES, unroll=16)
        def _row(o, bufa=bufa, bufb=bufb, a0=a0, c0=c0, a1=a1, c1=c1):
            t = tau_v[pl.ds(o, _LANES)]
            bufa[pl.ds(o, _LANES)] = c0 * jnp.exp(t * a0)
            bufb[pl.ds(o, _LANES)] = c1 * jnp.exp(t * a1)

        i0 = jnp.minimum(base + 2 * p, _WN - 1)
        i1 = jnp.minimum(base + 2 * p + 1, _WN - 1)
        handles[slot] = (
            pltpu.async_copy(bufa, out_hbm.at[i0], sems[slot]),
            pltpu.async_copy(bufb, out_hbm.at[i1], sems[slot]),
        )
    for hds in handles:
        for hd in hds:
            hd.wait()


@jax.jit
def _run(wf, par, tau):
    mesh = plsc.VectorSubcoreMesh(core_axis_name="c", subcore_axis_name="s")
    f = functools.partial(
        pl.kernel,
        mesh=mesh,
        out_type=jax.ShapeDtypeStruct((_WN, _K), jnp.float32),
        scratch_types=[
            pltpu.VMEM((_A * _RPT,), jnp.float32),
            pltpu.VMEM((_A + _LANES,), jnp.float32),
            pltpu.VMEM((_K,), jnp.float32),
            pltpu.VMEM((_K,), jnp.float32),
            pltpu.VMEM((_K,), jnp.float32),
            pltpu.VMEM((_K,), jnp.float32),
            pltpu.VMEM((_K,), jnp.float32),
            pltpu.VMEM((2 * _RPT,), jnp.float32),
            pltpu.SemaphoreType.DMA,
            pltpu.SemaphoreType.DMA,
            pltpu.SemaphoreType.DMA,
        ],
    )(_sc_body)
    return f(wf, par, tau)


def kernel(inputs, W, b):
    wf = inputs[:_WN, :_A]
    pad = jnp.broadcast_to(wf[_WN - 1], (_NTILES * _RPT - _WN, _A))
    wf = jnp.concatenate([wf, pad])
    wf = wf.reshape(_NTILES, _RPT, _A).transpose(0, 2, 1).reshape(_NTILES, _A * _RPT)
    tau = inputs[_WN:, :_L].reshape(_K)
    par = jnp.concatenate([W[:, 0], jnp.broadcast_to(b, (_LANES,))])
    out = _run(wf, par, tau)
    return out.reshape(_WN, _TN, _L)


# speedup vs baseline: 45.1305x; 6.3886x over previous
import functools
import math

import jax
import jax.numpy as jnp
from jax import lax
from jax.experimental import pallas as pl
from jax.experimental.pallas import tpu as pltpu
from jax.experimental.pallas import tpu_sc as plsc

_WN = 1000
_TN = 5000
_L = 4
_A = 128
_TP = 5120
_NB = _TP // 128
_RPW = _NB * _L
_LANES = 16
_NTILES = 32
_RPT = 32
_LN3 = math.log(3.0)


def _sc_body(wf_hbm, par_hbm, tau_hbm, out_hbm,
             wf_v, par_v, tau_v, row00, row01, row10, row11, ac_v,
             sem_in, sem0, sem1):
    cid = lax.axis_index("c")
    sid = lax.axis_index("s")
    wid = sid * 2 + cid
    base = wid * _RPT

    pltpu.sync_copy(par_hbm, par_v)
    pltpu.sync_copy(tau_hbm, tau_v)
    pltpu.sync_copy(wf_hbm.at[wid], wf_v)

    bvec = par_v[pl.ds(_A, _LANES)]

    for h in range(_RPT // _LANES):
        zvec = bvec
        for ch in range(_A // _LANES):
            pv = par_v[pl.ds(ch * _LANES, _LANES)]
            for j in range(_LANES):
                f = ch * _LANES + j
                zvec = zvec + wf_v[pl.ds(f * _RPT + h * _LANES, _LANES)] * pv[j]
        ac_v[pl.ds(h * _LANES, _LANES)] = zvec + _LN3
        ac_v[pl.ds(_RPT + h * _LANES, _LANES)] = 1.0 / (3.0 * (1.0 + jnp.exp(zvec)))

    bufs = ((row00, row01), (row10, row11))
    sems = (sem0, sem1)
    handles = [None, None]
    for p in range(_RPT // 2):
        slot = p % 2
        bufa, bufb = bufs[slot]
        if handles[slot] is not None:
            for hd in handles[slot]:
                hd.wait()
        sc = []
        for r in range(2):
            w = 2 * p + r
            h, ln = w // _LANES, w % _LANES
            av = ac_v[pl.ds(h * _LANES, _LANES)][ln]
            cv = ac_v[pl.ds(_RPT + h * _LANES, _LANES)][ln]
            sc.append((av, cv))
        (a0, c0), (a1, c1) = sc

        @plsc.parallel_loop(0, _RPW, 1, unroll=1)
        def _row(r, bufa=bufa, bufb=bufb, a0=a0, c0=c0, a1=a1, c1=c1):
            for cj in range(128 // _LANES):
                sl = pl.ds(cj * _LANES, _LANES)
                t = tau_v[r, sl]
                bufa[r, sl] = c0 * jnp.exp(t * a0)
                bufb[r, sl] = c1 * jnp.exp(t * a1)

        i0 = jnp.minimum(base + 2 * p, _WN - 1)
        i1 = jnp.minimum(base + 2 * p + 1, _WN - 1)
        handles[slot] = (
            pltpu.async_copy(bufa, out_hbm.at[pl.ds(i0 * _RPW, _RPW)], sems[slot]),
            pltpu.async_copy(bufb, out_hbm.at[pl.ds(i1 * _RPW, _RPW)], sems[slot]),
        )
    for hds in handles:
        for hd in hds:
            hd.wait()


@jax.jit
def _run(wf, par, tau):
    mesh = plsc.VectorSubcoreMesh(core_axis_name="c", subcore_axis_name="s")
    f = functools.partial(
        pl.kernel,
        mesh=mesh,
        out_type=jax.ShapeDtypeStruct((_WN * _RPW, 128), jnp.float32),
        scratch_types=[
            pltpu.VMEM((_A * _RPT,), jnp.float32),
            pltpu.VMEM((_A + _LANES,), jnp.float32),
            pltpu.VMEM((_RPW, 128), jnp.float32),
            pltpu.VMEM((_RPW, 128), jnp.float32),
            pltpu.VMEM((_RPW, 128), jnp.float32),
            pltpu.VMEM((_RPW, 128), jnp.float32),
            pltpu.VMEM((_RPW, 128), jnp.float32),
            pltpu.VMEM((2 * _RPT,), jnp.float32),
            pltpu.SemaphoreType.DMA,
            pltpu.SemaphoreType.DMA,
            pltpu.SemaphoreType.DMA,
        ],
    )(_sc_body)
    return f(wf, par, tau)


def kernel(inputs, W, b):
    wf = inputs[:_WN, :_A]
    pad = jnp.broadcast_to(wf[_WN - 1], (_NTILES * _RPT - _WN, _A))
    wf = jnp.concatenate([wf, pad])
    wf = wf.reshape(_NTILES, _RPT, _A).transpose(0, 2, 1).reshape(_NTILES, _A * _RPT)
    tau2 = jnp.pad(inputs[_WN:, :_L], ((0, _TP - _TN), (0, 0)))
    tau_p = tau2.reshape(_NB, 128, _L).transpose(0, 2, 1).reshape(_RPW, 128)
    par = jnp.concatenate([W[:, 0], jnp.broadcast_to(b, (_LANES,))])
    out = _run(wf, par, tau_p)
    out = out.reshape(_WN, _NB, _L, 128).transpose(0, 1, 3, 2)
    return out.reshape(_WN, _TP, _L)[:, :_TN, :]
